# baseline (speedup 1.0000x reference)
import jax
import jax.numpy as jnp
from jax import lax
from jax.experimental import pallas as pl
from jax.experimental.pallas import tpu as pltpu

B = 32
N = 512 * 512
SPB = 8
GRID = B // SPB


def _tc_body(x_ref, t_ref, out_ref):
    i = pl.program_id(0)
    x = x_ref[...]
    t = t_ref[...].astype(jnp.float32)

    sig = 0.5 + 0.5 * jnp.tanh(x * 0.5)
    bce = jnp.maximum(x, 0.0) - x * t + jnp.log1p(jnp.exp(-jnp.abs(x)))

    for u in range(SPB):
        out_ref[i * SPB + u, 0] = jnp.sum(sig[u])
        out_ref[i * SPB + u, 1] = jnp.sum(sig[u] * t[u])
        out_ref[i * SPB + u, 2] = jnp.sum(t[u])
        out_ref[i * SPB + u, 3] = jnp.sum(bce[u])


def _tc_partials(x3, target):
    return pl.pallas_call(
        _tc_body,
        grid=(GRID,),
        in_specs=[
            pl.BlockSpec((SPB, 512, 512), lambda i: (i, 0, 0)),
            pl.BlockSpec((SPB, 512, 512), lambda i: (i, 0, 0)),
        ],
        out_specs=pl.BlockSpec(memory_space=pltpu.SMEM),
        out_shape=jax.ShapeDtypeStruct((B, 4), jnp.float32),
        compiler_params=pltpu.CompilerParams(
            dimension_semantics=("arbitrary",),
            vmem_limit_bytes=110 * 1024 * 1024,
        ),
    )(x3, target)


@jax.jit
def kernel(inputs, target):
    x3 = inputs.reshape(B, 512, 512)
    parts = _tc_partials(x3, target)
    s_sum = parts[:, 0]
    st_sum = parts[:, 1]
    t_sum = parts[:, 2]
    b_sum = parts[:, 3]
    dice = 1.0 - (2.0 * st_sum + 1.0) / (s_sum + t_sum + 1.0)
    loss = jnp.mean(dice) + jnp.sum(b_sum) / (B * N)
    return loss.reshape(1)


# speedup vs baseline: 1.0504x; 1.0504x over previous
import jax
import jax.numpy as jnp
from jax import lax
from jax.experimental import pallas as pl
from jax.experimental.pallas import tpu as pltpu

B = 32
N = 512 * 512
SPB = 4
GRID = B // SPB


def _tc_body(x_ref, t_ref, out_ref):
    i = pl.program_id(0)
    x = x_ref[...]
    t = t_ref[...].astype(jnp.float32)

    sig = 0.5 + 0.5 * jnp.tanh(x * 0.5)
    bce = jnp.maximum(x, 0.0) - x * t + jnp.log1p(jnp.exp(-jnp.abs(x)))

    for u in range(SPB):
        out_ref[i * SPB + u, 0] = jnp.sum(sig[u])
        out_ref[i * SPB + u, 1] = jnp.sum(sig[u] * t[u])
        out_ref[i * SPB + u, 2] = jnp.sum(t[u])
        out_ref[i * SPB + u, 3] = jnp.sum(bce[u])


def _tc_partials(x3, target):
    return pl.pallas_call(
        _tc_body,
        grid=(GRID,),
        in_specs=[
            pl.BlockSpec((SPB, 512, 512), lambda i: (i, 0, 0)),
            pl.BlockSpec((SPB, 512, 512), lambda i: (i, 0, 0)),
        ],
        out_specs=pl.BlockSpec(memory_space=pltpu.SMEM),
        out_shape=jax.ShapeDtypeStruct((B, 4), jnp.float32),
        compiler_params=pltpu.CompilerParams(
            dimension_semantics=("arbitrary",),
        ),
    )(x3, target)


@jax.jit
def kernel(inputs, target):
    x3 = inputs.reshape(B, 512, 512)
    parts = _tc_partials(x3, target)
    s_sum = parts[:, 0]
    st_sum = parts[:, 1]
    t_sum = parts[:, 2]
    b_sum = parts[:, 3]
    dice = 1.0 - (2.0 * st_sum + 1.0) / (s_sum + t_sum + 1.0)
    loss = jnp.mean(dice) + jnp.sum(b_sum) / (B * N)
    return loss.reshape(1)


# speedup vs baseline: 1.1197x; 1.0660x over previous
import jax
import jax.numpy as jnp
from jax import lax
from jax.experimental import pallas as pl
from jax.experimental.pallas import tpu as pltpu

B = 32
N = 512 * 512
SPB = 4
GRID = B // SPB


def _tc_body(x_ref, t_ref, out_ref):
    i = pl.program_id(0)
    x = x_ref[...]
    t = t_ref[...].astype(jnp.float32)

    sig = 0.5 + 0.5 * jnp.tanh(x * 0.5)
    mx = jnp.maximum(x, 0.0)
    lp = jnp.log1p(jnp.exp(-jnp.abs(x)))
    xt = x * t

    for u in range(SPB):
        out_ref[i * SPB + u, 0] = jnp.sum(sig[u])
        out_ref[i * SPB + u, 1] = jnp.sum(sig[u] * t[u])
        out_ref[i * SPB + u, 2] = jnp.sum(t[u])
        out_ref[i * SPB + u, 3] = (jnp.sum(mx[u]) - jnp.sum(xt[u])
                                   + jnp.sum(lp[u]))


def _tc_partials(x3, target):
    return pl.pallas_call(
        _tc_body,
        grid=(GRID,),
        in_specs=[
            pl.BlockSpec((SPB, 512, 512), lambda i: (i, 0, 0)),
            pl.BlockSpec((SPB, 512, 512), lambda i: (i, 0, 0)),
        ],
        out_specs=pl.BlockSpec(memory_space=pltpu.SMEM),
        out_shape=jax.ShapeDtypeStruct((B, 4), jnp.float32),
        compiler_params=pltpu.CompilerParams(
            dimension_semantics=("arbitrary",),
        ),
    )(x3, target)


@jax.jit
def kernel(inputs, target):
    x3 = inputs.reshape(B, 512, 512)
    parts = _tc_partials(x3, target)
    s_sum = parts[:, 0]
    st_sum = parts[:, 1]
    t_sum = parts[:, 2]
    b_sum = parts[:, 3]
    dice = 1.0 - (2.0 * st_sum + 1.0) / (s_sum + t_sum + 1.0)
    loss = jnp.mean(dice) + jnp.sum(b_sum) / (B * N)
    return loss.reshape(1)


# speedup vs baseline: 1.3303x; 1.1880x over previous
import jax
import jax.numpy as jnp
from jax import lax
from jax.experimental import pallas as pl
from jax.experimental.pallas import tpu as pltpu

B = 32
N = 512 * 512
SPB = 4
GRID = B // SPB


def _tc_body(x_ref, t_ref, out_ref):
    i = pl.program_id(0)
    x = x_ref[...]
    t = t_ref[...].astype(jnp.float32)

    th = jnp.tanh(x * 0.5)
    sig = 0.5 + 0.5 * th
    mx = jnp.maximum(x, 0.0)
    l1t = jnp.log1p(jnp.abs(th))
    xt = x * t

    for u in range(SPB):
        out_ref[i * SPB + u, 0] = jnp.sum(sig[u])
        out_ref[i * SPB + u, 1] = jnp.sum(sig[u] * t[u])
        out_ref[i * SPB + u, 2] = jnp.sum(t[u])
        out_ref[i * SPB + u, 3] = (jnp.sum(mx[u]) - jnp.sum(xt[u])
                                   - jnp.sum(l1t[u]))


def _tc_partials(x3, target):
    return pl.pallas_call(
        _tc_body,
        grid=(GRID,),
        in_specs=[
            pl.BlockSpec((SPB, 512, 512), lambda i: (i, 0, 0)),
            pl.BlockSpec((SPB, 512, 512), lambda i: (i, 0, 0)),
        ],
        out_specs=pl.BlockSpec(memory_space=pltpu.SMEM),
        out_shape=jax.ShapeDtypeStruct((B, 4), jnp.float32),
        compiler_params=pltpu.CompilerParams(
            dimension_semantics=("arbitrary",),
        ),
    )(x3, target)


@jax.jit
def kernel(inputs, target):
    x3 = inputs.reshape(B, 512, 512)
    parts = _tc_partials(x3, target)
    s_sum = parts[:, 0]
    st_sum = parts[:, 1]
    t_sum = parts[:, 2]
    b_sum = parts[:, 3] + N * 0.6931471805599453
    dice = 1.0 - (2.0 * st_sum + 1.0) / (s_sum + t_sum + 1.0)
    loss = jnp.mean(dice) + jnp.sum(b_sum) / (B * N)
    return loss.reshape(1)
